# baseline (speedup 1.0000x reference)
import functools
import math

import jax
import jax.numpy as jnp
from jax import lax
from jax.experimental import pallas as pl
from jax.experimental.pallas import tpu as pltpu
from jax.experimental.pallas import tpu_sc as plsc
import numpy as np

_SS = np.array([[64, 64], [32, 32], [16, 16], [8, 8]], dtype=np.int32)
_BS, _NQ, _D, _H, _L, _P = 2, 5440, 256, 8, 4, 4
_NV = int((_SS[:, 0] * _SS[:, 1]).sum())
_CH = _D // _H
_LSI = np.concatenate([[0], np.cumsum(_SS[:, 0] * _SS[:, 1])[:-1]]).astype(np.int32)

_ROWS = _BS * _NQ * _H
_NSAMP = _ROWS * _L * _P
_NWORK = 32
_RPW = _ROWS // _NWORK
_RCHUNK = 8
_NCHUNK = _RPW // _RCHUNK

_MM_BLK = 1088


def _mm_bias_kernel(x_ref, w_ref, b_ref, o_ref):
    o_ref[...] = (
        jnp.dot(x_ref[...], w_ref[...], preferred_element_type=jnp.float32)
        + b_ref[...]
    ).astype(o_ref.dtype)


def _mm_bias_res_kernel(x_ref, w_ref, b_ref, r_ref, o_ref):
    o_ref[...] = (
        jnp.dot(x_ref[...], w_ref[...], preferred_element_type=jnp.float32)
        + b_ref[...]
        + r_ref[...]
    )


def _matmul_bias(x, w_t, b, residual=None, out_dtype=jnp.float32):
    n, _ = x.shape
    m = w_t.shape[1]
    grid = (n // _MM_BLK,)
    in_specs = [
        pl.BlockSpec((_MM_BLK, x.shape[1]), lambda i: (i, 0)),
        pl.BlockSpec(w_t.shape, lambda i: (0, 0)),
        pl.BlockSpec((1, m), lambda i: (0, 0)),
    ]
    args = [x, w_t, b.reshape(1, m)]
    kern = _mm_bias_kernel
    if residual is not None:
        in_specs.append(pl.BlockSpec((_MM_BLK, m), lambda i: (i, 0)))
        args.append(residual)
        kern = _mm_bias_res_kernel
    return pl.pallas_call(
        kern,
        grid=grid,
        in_specs=in_specs,
        out_specs=pl.BlockSpec((_MM_BLK, m), lambda i: (i, 0)),
        out_shape=jax.ShapeDtypeStruct((n, m), out_dtype),
    )(*args)


_QB = 544


def _prep_kernel(q_ref, v_ref, wv_ref, bv_ref, wox_ref, wat_ref,
                 box_ref, boy_ref, bat_ref, rp_ref, gones_ref,
                 idx_ref, wgt_ref, tbl_ref):
    b = pl.program_id(0)
    q = q_ref[0]

    tbl_ref[0] = (
        jnp.dot(v_ref[0], wv_ref[...], preferred_element_type=jnp.float32)
        + bv_ref[...]
    ).astype(jnp.bfloat16)

    so_xy = jnp.dot(q, wox_ref[...], preferred_element_type=jnp.float32)
    so_x = so_xy[:, :128] + box_ref[...]
    so_y = so_xy[:, 128:] + boy_ref[...]
    logits = jnp.dot(q, wat_ref[...], preferred_element_type=jnp.float32) + bat_ref[...]

    ex = jnp.exp(logits)
    gsum = jnp.dot(ex, gones_ref[...], preferred_element_type=jnp.float32)
    aw = ex / gsum

    lane = lax.broadcasted_iota(jnp.int32, (_QB, 128), 1)
    lsel = (lane // 4) % 4
    hsel = lane // 16

    def sel(vals, dtype):
        out = jnp.full(lane.shape, vals[3], dtype)
        for i in (2, 1, 0):
            out = jnp.where(lsel == i, dtype(vals[i]), out)
        return out

    wl_f = sel([float(_SS[i, 1]) for i in range(4)], jnp.float32)
    hl_f = sel([float(_SS[i, 0]) for i in range(4)], jnp.float32)
    wl_i = sel([int(_SS[i, 1]) for i in range(4)], jnp.int32)
    ls_i = sel([int(_LSI[i]) for i in range(4)], jnp.int32)

    rp = rp_ref[0]
    rx = jnp.zeros((_QB, 128), jnp.float32)
    ry = jnp.zeros((_QB, 128), jnp.float32)
    for l in range(_L):
        rx = jnp.where(lsel == l, rp[:, 2 * l:2 * l + 1], rx)
        ry = jnp.where(lsel == l, rp[:, 2 * l + 1:2 * l + 2], ry)

    gx = rx * wl_f - 0.5 + so_x
    gy = ry * hl_f - 0.5 + so_y
    x0 = jnp.floor(gx)
    y0 = jnp.floor(gy)
    wx1 = gx - x0
    wx0 = 1.0 - wx1
    wy1 = gy - y0
    wy0 = 1.0 - wy1

    base_b = b * _NV

    for c, (dx, dy, wx, wy) in enumerate(
        ((0, 0, wx0, wy0), (1, 0, wx1, wy0), (0, 1, wx0, wy1), (1, 1, wx1, wy1))
    ):
        xi = x0 + dx
        yi = y0 + dy
        valid = (
            (xi >= 0.0) & (xi <= wl_f - 1.0) & (yi >= 0.0) & (yi <= hl_f - 1.0)
        )
        xc = jnp.clip(xi, 0.0, wl_f - 1.0).astype(jnp.int32)
        yc = jnp.clip(yi, 0.0, hl_f - 1.0).astype(jnp.int32)
        row = (base_b + ls_i + yc * wl_i + xc) * _H + hsel
        w = aw * wx * wy * valid.astype(jnp.float32)
        idx_ref[c, 0] = row
        w16 = lax.bitcast_convert_type(w.astype(jnp.bfloat16), jnp.uint16)
        w32 = w16.astype(jnp.uint32)
        wgt_ref[c, 0] = lax.bitcast_convert_type(w32 | (w32 << 16), jnp.int32)


def _prep(query, value, wv_t, bv, woxy_t, wat_t, box, boy, bat, rp):
    grid = (_BS, _NQ // _QB)
    full = lambda shape: pl.BlockSpec(shape, lambda b, i: tuple(0 for _ in shape))
    in_specs = [
        pl.BlockSpec((1, _QB, _D), lambda b, i: (b, i, 0)),
        pl.BlockSpec((1, _QB, _D), lambda b, i: (b, i, 0)),
        full(wv_t.shape), full((1, _D)),
        full(woxy_t.shape), full(wat_t.shape),
        full((1, 128)), full((1, 128)), full((1, 128)),
        pl.BlockSpec((1, _QB, 2 * _L), lambda b, i: (b, i, 0)),
        full((128, 128)),
    ]
    out_specs = [
        pl.BlockSpec((4, 1, _QB, 128), lambda b, i: (0, b, i, 0)),
        pl.BlockSpec((4, 1, _QB, 128), lambda b, i: (0, b, i, 0)),
        pl.BlockSpec((1, _QB, _D), lambda b, i: (b, i, 0)),
    ]
    pc = pl.pallas_call(
        _prep_kernel,
        grid=grid,
        in_specs=in_specs,
        out_specs=out_specs,
        out_shape=[
            jax.ShapeDtypeStruct((4, _BS, _NQ, 128), jnp.int32),
            jax.ShapeDtypeStruct((4, _BS, _NQ, 128), jnp.int32),
            jax.ShapeDtypeStruct((_BS, _NV, _D), jnp.bfloat16),
        ],
    )
    gones = jnp.asarray(np.kron(np.eye(_H, dtype=np.float32),
                                np.ones((16, 16), np.float32)))
    return pc(query, value, wv_t, bv.reshape(1, _D), woxy_t, wat_t,
              box.reshape(1, 128), boy.reshape(1, 128), bat.reshape(1, 128),
              rp, gones)


_GRP = 10
_NGRP = _NCHUNK // _GRP
_GRP_R = _GRP * _RCHUNK
_CPW = _NCHUNK


def _sc_gather_accum(table, idx, wgt):
    mesh = plsc.VectorSubcoreMesh(core_axis_name="c", subcore_axis_name="s")
    info = plsc.get_sparse_core_info()
    nc = info.num_cores

    @functools.partial(
        pl.kernel,
        mesh=mesh,
        out_type=jax.ShapeDtypeStruct((_ROWS, _CH), jnp.float32),
        compiler_params=pltpu.CompilerParams(
            use_tc_tiling_on_sc=False, needs_layout_passes=False),
        scratch_types=[
            pltpu.VMEM((2, 4, _GRP, 128), jnp.int32),
            pltpu.VMEM((2, 4, _GRP, 128), jnp.int32),
            pltpu.VMEM((2, 4, 128, _CH), jnp.bfloat16),
            pltpu.VMEM((2, _GRP_R, _CH), jnp.float32),
            pltpu.VMEM_SHARED((_ROWS, _CH), jnp.bfloat16),
            pltpu.SemaphoreType.DMA,
            pltpu.SemaphoreType.DMA,
            pltpu.SemaphoreType.DMA,
            pltpu.SemaphoreType.DMA,
            pltpu.SemaphoreType.DMA,
            pltpu.SemaphoreType.DMA,
        ],
    )
    def k(table_hbm, idx_hbm, wgt_hbm, out_hbm, idx_g, wgt_g, rows_v, out_g,
          table_sh, si0, si1, so0, so1, sb0, sb1):
        wid = lax.axis_index("s") * nc + lax.axis_index("c")
        sid = lax.axis_index("s")
        nst = _ROWS // 16
        pltpu.sync_copy(table_hbm.at[pl.ds(sid * nst, nst)],
                        table_sh.at[pl.ds(sid * nst, nst)])
        plsc.subcore_barrier()
        isems = (si0, si1)
        osems = (so0, so1)
        gsems = (sb0, sb1)

        def ifire(g, s):
            base_c = wid * _CPW + g * _GRP
            for c in range(4):
                pltpu.async_copy(
                    idx_hbm.at[c, pl.ds(base_c, _GRP)], idx_g.at[s, c], isems[s])
                pltpu.async_copy(
                    wgt_hbm.at[c, pl.ds(base_c, _GRP)], wgt_g.at[s, c], isems[s])

        def iwait(s):
            for c in range(4):
                pltpu.make_async_copy(
                    idx_hbm.at[c, pl.ds(0, _GRP)], idx_g.at[s, c], isems[s]).wait()
                pltpu.make_async_copy(
                    wgt_hbm.at[c, pl.ds(0, _GRP)], wgt_g.at[s, c], isems[s]).wait()

        def gfire(off, slot, s):
            for c in range(4):
                pltpu.async_copy(
                    table_sh.at[idx_g.at[s, c, off]],
                    rows_v.at[slot, c],
                    gsems[slot],
                )

        def gwait(slot, s):
            for c in range(4):
                pltpu.make_async_copy(
                    table_sh.at[idx_g.at[s, c, 0]],
                    rows_v.at[slot, c],
                    gsems[slot],
                ).wait()

        def compute(off, slot, s):
            def pair_body(t, carry2):
                for r in (t * 2, t * 2 + 1):
                    _one_row(off, slot, s, r)
                return carry2

            def _one_row(off, slot, s, r):
                j0 = r * (_L * _P)
                wvs = [wgt_g[s, c, off, pl.ds(j0, 16)] for c in range(4)]
                e_accs = [jnp.zeros((16,), jnp.float32) for _ in range(2)]
                o_accs = [jnp.zeros((16,), jnp.float32) for _ in range(2)]
                for j in range(_L * _P):
                    ps = []
                    for c in range(4):
                        w = wvs[c][j]
                        wspl = jnp.full((16,), w, jnp.int32)
                        wbf = plsc.bitcast(wspl, jnp.bfloat16)
                        ps.append(wbf * rows_v[slot, c, j0 + j])
                    p = (ps[0] + ps[1]) + (ps[2] + ps[3])
                    ev, ov = plsc.unpack(p, format=plsc.PackFormat.INTERLEAVED)
                    e_accs[j % 2] = e_accs[j % 2] + ev
                    o_accs[j % 2] = o_accs[j % 2] + ov
                acc0 = e_accs[0] + e_accs[1]
                acc1 = o_accs[0] + o_accs[1]
                out_g[s, off * _RCHUNK + r, pl.ds(0, 16)] = acc0
                out_g[s, off * _RCHUNK + r, pl.ds(16, 16)] = acc1

            lax.fori_loop(0, _RCHUNK // 2, pair_body, 0)

        def process_group(g, s):
            base_r = wid * _RPW + g * _GRP_R

            @pl.when(g >= 2)
            def _():
                pltpu.make_async_copy(
                    out_g.at[s], out_hbm.at[pl.ds(0, _GRP_R)], osems[s]).wait()

            def step(cc, carry2):
                off_a = cc * 2
                gfire(off_a + 1, 1, s)
                gwait(0, s)
                compute(off_a, 0, s)

                @pl.when(cc < _GRP // 2 - 1)
                def _():
                    gfire(off_a + 2, 0, s)

                @pl.when((cc == _GRP // 2 - 1) & (g + 1 < _NGRP))
                def _():
                    iwait(1 - s)
                    gfire(0, 0, 1 - s)

                gwait(1, s)
                compute(off_a + 1, 1, s)
                return carry2

            lax.fori_loop(0, _GRP // 2, step, 0)
            pltpu.async_copy(out_g.at[s], out_hbm.at[pl.ds(base_r, _GRP_R)],
                             osems[s])

            @pl.when(g + 2 < _NGRP)
            def _():
                ifire(g + 2, s)

        ifire(0, 0)
        ifire(1, 1)
        iwait(0)
        gfire(0, 0, 0)

        def outer(gg, carry):
            process_group(gg * 2, 0)
            process_group(gg * 2 + 1, 1)
            return carry

        lax.fori_loop(0, _NGRP // 2, outer, 0)
        for s in range(2):
            pltpu.make_async_copy(
                out_g.at[s], out_hbm.at[pl.ds(0, _GRP_R)], osems[s]).wait()

    return k(table, idx, wgt)


def kernel(query, value, reference_points, spatial_shapes, level_start_index,
           W_value, b_value, W_offsets, b_offsets, W_attn, b_attn, W_out, b_out):
    qf = query.reshape(_BS * _NQ, _D)

    woxy_t = jnp.concatenate([W_offsets[0::2], W_offsets[1::2]], axis=0).T
    idx, wgt, table = _prep(
        query, value, W_value.T, b_value,
        woxy_t, W_attn.T,
        b_offsets[0::2], b_offsets[1::2], b_attn,
        reference_points.reshape(_BS, _NQ, 2 * _L),
    )
    idx = idx.reshape(4, _NSAMP // 128, 128)
    wgt = wgt.reshape(4, _NSAMP // 128, 128)
    table = table.reshape(_ROWS, _CH)

    msda = _sc_gather_accum(table, idx, wgt).reshape(_BS * _NQ, _D)

    p = np.arange(32)
    chan = np.where(p < 16, 2 * p, 2 * (p - 16) + 1)
    perm = (np.arange(_H)[:, None] * 32 + chan[None, :]).reshape(-1)
    out = _matmul_bias(msda, W_out.T[perm], b_out, residual=qf)
    return out.reshape(_BS, _NQ, _D)


# speedup vs baseline: 1.0127x; 1.0127x over previous
import functools
import math

import jax
import jax.numpy as jnp
from jax import lax
from jax.experimental import pallas as pl
from jax.experimental.pallas import tpu as pltpu
from jax.experimental.pallas import tpu_sc as plsc
import numpy as np

_SS = np.array([[64, 64], [32, 32], [16, 16], [8, 8]], dtype=np.int32)
_BS, _NQ, _D, _H, _L, _P = 2, 5440, 256, 8, 4, 4
_NV = int((_SS[:, 0] * _SS[:, 1]).sum())
_CH = _D // _H
_LSI = np.concatenate([[0], np.cumsum(_SS[:, 0] * _SS[:, 1])[:-1]]).astype(np.int32)

_ROWS = _BS * _NQ * _H
_NSAMP = _ROWS * _L * _P
_NWORK = 32
_RPW = _ROWS // _NWORK
_RCHUNK = 8
_NCHUNK = _RPW // _RCHUNK

_MM_BLK = 1088


def _mm_bias_kernel(x_ref, w_ref, b_ref, o_ref):
    o_ref[...] = (
        jnp.dot(x_ref[...], w_ref[...], preferred_element_type=jnp.float32)
        + b_ref[...]
    ).astype(o_ref.dtype)


def _mm_bias_res_kernel(x_ref, w_ref, b_ref, r_ref, o_ref):
    o_ref[...] = (
        jnp.dot(x_ref[...], w_ref[...], preferred_element_type=jnp.float32)
        + b_ref[...]
        + r_ref[...]
    )


def _matmul_bias(x, w_t, b, residual=None, out_dtype=jnp.float32):
    n, _ = x.shape
    m = w_t.shape[1]
    grid = (n // _MM_BLK,)
    in_specs = [
        pl.BlockSpec((_MM_BLK, x.shape[1]), lambda i: (i, 0)),
        pl.BlockSpec(w_t.shape, lambda i: (0, 0)),
        pl.BlockSpec((1, m), lambda i: (0, 0)),
    ]
    args = [x, w_t, b.reshape(1, m)]
    kern = _mm_bias_kernel
    if residual is not None:
        in_specs.append(pl.BlockSpec((_MM_BLK, m), lambda i: (i, 0)))
        args.append(residual)
        kern = _mm_bias_res_kernel
    return pl.pallas_call(
        kern,
        grid=grid,
        in_specs=in_specs,
        out_specs=pl.BlockSpec((_MM_BLK, m), lambda i: (i, 0)),
        out_shape=jax.ShapeDtypeStruct((n, m), out_dtype),
    )(*args)


_QB = 1088


def _prep_kernel(q_ref, v_ref, wv_ref, bv_ref, wox_ref, wat_ref,
                 box_ref, boy_ref, bat_ref, rp_ref, gones_ref,
                 idx_ref, wgt_ref, tbl_ref):
    b = pl.program_id(0)
    q = q_ref[0]

    tbl_ref[0] = (
        jnp.dot(v_ref[0], wv_ref[...], preferred_element_type=jnp.float32)
        + bv_ref[...]
    ).astype(jnp.bfloat16)

    so_xy = jnp.dot(q, wox_ref[...], preferred_element_type=jnp.float32)
    so_x = so_xy[:, :128] + box_ref[...]
    so_y = so_xy[:, 128:] + boy_ref[...]
    logits = jnp.dot(q, wat_ref[...], preferred_element_type=jnp.float32) + bat_ref[...]

    ex = jnp.exp(logits)
    gsum = jnp.dot(ex, gones_ref[...], preferred_element_type=jnp.float32)
    aw = ex / gsum

    lane = lax.broadcasted_iota(jnp.int32, (_QB, 128), 1)
    lsel = (lane // 4) % 4
    hsel = lane // 16

    def sel(vals, dtype):
        out = jnp.full(lane.shape, vals[3], dtype)
        for i in (2, 1, 0):
            out = jnp.where(lsel == i, dtype(vals[i]), out)
        return out

    wl_f = sel([float(_SS[i, 1]) for i in range(4)], jnp.float32)
    hl_f = sel([float(_SS[i, 0]) for i in range(4)], jnp.float32)
    wl_i = sel([int(_SS[i, 1]) for i in range(4)], jnp.int32)
    ls_i = sel([int(_LSI[i]) for i in range(4)], jnp.int32)

    rp = rp_ref[0]
    rx = jnp.zeros((_QB, 128), jnp.float32)
    ry = jnp.zeros((_QB, 128), jnp.float32)
    for l in range(_L):
        rx = jnp.where(lsel == l, rp[:, 2 * l:2 * l + 1], rx)
        ry = jnp.where(lsel == l, rp[:, 2 * l + 1:2 * l + 2], ry)

    gx = rx * wl_f - 0.5 + so_x
    gy = ry * hl_f - 0.5 + so_y
    x0 = jnp.floor(gx)
    y0 = jnp.floor(gy)
    wx1 = gx - x0
    wx0 = 1.0 - wx1
    wy1 = gy - y0
    wy0 = 1.0 - wy1

    base_b = b * _NV

    for c, (dx, dy, wx, wy) in enumerate(
        ((0, 0, wx0, wy0), (1, 0, wx1, wy0), (0, 1, wx0, wy1), (1, 1, wx1, wy1))
    ):
        xi = x0 + dx
        yi = y0 + dy
        valid = (
            (xi >= 0.0) & (xi <= wl_f - 1.0) & (yi >= 0.0) & (yi <= hl_f - 1.0)
        )
        xc = jnp.clip(xi, 0.0, wl_f - 1.0).astype(jnp.int32)
        yc = jnp.clip(yi, 0.0, hl_f - 1.0).astype(jnp.int32)
        row = (base_b + ls_i + yc * wl_i + xc) * _H + hsel
        w = aw * wx * wy * valid.astype(jnp.float32)
        idx_ref[c, 0] = row
        w16 = lax.bitcast_convert_type(w.astype(jnp.bfloat16), jnp.uint16)
        w32 = w16.astype(jnp.uint32)
        wgt_ref[c, 0] = lax.bitcast_convert_type(w32 | (w32 << 16), jnp.int32)


def _prep(query, value, wv_t, bv, woxy_t, wat_t, box, boy, bat, rp):
    grid = (_BS, _NQ // _QB)
    full = lambda shape: pl.BlockSpec(shape, lambda b, i: tuple(0 for _ in shape))
    in_specs = [
        pl.BlockSpec((1, _QB, _D), lambda b, i: (b, i, 0)),
        pl.BlockSpec((1, _QB, _D), lambda b, i: (b, i, 0)),
        full(wv_t.shape), full((1, _D)),
        full(woxy_t.shape), full(wat_t.shape),
        full((1, 128)), full((1, 128)), full((1, 128)),
        pl.BlockSpec((1, _QB, 2 * _L), lambda b, i: (b, i, 0)),
        full((128, 128)),
    ]
    out_specs = [
        pl.BlockSpec((4, 1, _QB, 128), lambda b, i: (0, b, i, 0)),
        pl.BlockSpec((4, 1, _QB, 128), lambda b, i: (0, b, i, 0)),
        pl.BlockSpec((1, _QB, _D), lambda b, i: (b, i, 0)),
    ]
    pc = pl.pallas_call(
        _prep_kernel,
        grid=grid,
        in_specs=in_specs,
        out_specs=out_specs,
        out_shape=[
            jax.ShapeDtypeStruct((4, _BS, _NQ, 128), jnp.int32),
            jax.ShapeDtypeStruct((4, _BS, _NQ, 128), jnp.int32),
            jax.ShapeDtypeStruct((_BS, _NV, _D), jnp.bfloat16),
        ],
    )
    gones = jnp.asarray(np.kron(np.eye(_H, dtype=np.float32),
                                np.ones((16, 16), np.float32)))
    return pc(query, value, wv_t, bv.reshape(1, _D), woxy_t, wat_t,
              box.reshape(1, 128), boy.reshape(1, 128), bat.reshape(1, 128),
              rp, gones)


_GRP = 10
_NGRP = _NCHUNK // _GRP
_GRP_R = _GRP * _RCHUNK
_CPW = _NCHUNK


def _sc_gather_accum(table, idx, wgt):
    mesh = plsc.VectorSubcoreMesh(core_axis_name="c", subcore_axis_name="s")
    info = plsc.get_sparse_core_info()
    nc = info.num_cores

    @functools.partial(
        pl.kernel,
        mesh=mesh,
        out_type=jax.ShapeDtypeStruct((_ROWS, _CH), jnp.float32),
        compiler_params=pltpu.CompilerParams(
            use_tc_tiling_on_sc=False, needs_layout_passes=False),
        scratch_types=[
            pltpu.VMEM((2, 4, _GRP, 128), jnp.int32),
            pltpu.VMEM((2, 4, _GRP, 128), jnp.int32),
            pltpu.VMEM((2, 4, 128, _CH), jnp.bfloat16),
            pltpu.VMEM((2, _GRP_R, _CH), jnp.float32),
            pltpu.VMEM_SHARED((_ROWS, _CH), jnp.bfloat16),
            pltpu.SemaphoreType.DMA,
            pltpu.SemaphoreType.DMA,
            pltpu.SemaphoreType.DMA,
            pltpu.SemaphoreType.DMA,
            pltpu.SemaphoreType.DMA,
            pltpu.SemaphoreType.DMA,
        ],
    )
    def k(table_hbm, idx_hbm, wgt_hbm, out_hbm, idx_g, wgt_g, rows_v, out_g,
          table_sh, si0, si1, so0, so1, sb0, sb1):
        wid = lax.axis_index("s") * nc + lax.axis_index("c")
        sid = lax.axis_index("s")
        nst = _ROWS // 16
        pltpu.sync_copy(table_hbm.at[pl.ds(sid * nst, nst)],
                        table_sh.at[pl.ds(sid * nst, nst)])
        plsc.subcore_barrier()
        isems = (si0, si1)
        osems = (so0, so1)
        gsems = (sb0, sb1)

        def ifire(g, s):
            base_c = wid * _CPW + g * _GRP
            for c in range(4):
                pltpu.async_copy(
                    idx_hbm.at[c, pl.ds(base_c, _GRP)], idx_g.at[s, c], isems[s])
                pltpu.async_copy(
                    wgt_hbm.at[c, pl.ds(base_c, _GRP)], wgt_g.at[s, c], isems[s])

        def iwait(s):
            for c in range(4):
                pltpu.make_async_copy(
                    idx_hbm.at[c, pl.ds(0, _GRP)], idx_g.at[s, c], isems[s]).wait()
                pltpu.make_async_copy(
                    wgt_hbm.at[c, pl.ds(0, _GRP)], wgt_g.at[s, c], isems[s]).wait()

        def gfire(off, slot, s):
            for c in range(4):
                pltpu.async_copy(
                    table_sh.at[idx_g.at[s, c, off]],
                    rows_v.at[slot, c],
                    gsems[slot],
                )

        def gwait(slot, s):
            for c in range(4):
                pltpu.make_async_copy(
                    table_sh.at[idx_g.at[s, c, 0]],
                    rows_v.at[slot, c],
                    gsems[slot],
                ).wait()

        def compute(off, slot, s):
            def pair_body(t, carry2):
                for r in (t * 2, t * 2 + 1):
                    _one_row(off, slot, s, r)
                return carry2

            def _one_row(off, slot, s, r):
                j0 = r * (_L * _P)
                wvs = [wgt_g[s, c, off, pl.ds(j0, 16)] for c in range(4)]
                e_accs = [jnp.zeros((16,), jnp.float32) for _ in range(2)]
                o_accs = [jnp.zeros((16,), jnp.float32) for _ in range(2)]
                for j in range(_L * _P):
                    ps = []
                    for c in range(4):
                        w = wvs[c][j]
                        wspl = jnp.full((16,), w, jnp.int32)
                        wbf = plsc.bitcast(wspl, jnp.bfloat16)
                        ps.append(wbf * rows_v[slot, c, j0 + j])
                    p = (ps[0] + ps[1]) + (ps[2] + ps[3])
                    ev, ov = plsc.unpack(p, format=plsc.PackFormat.INTERLEAVED)
                    e_accs[j % 2] = e_accs[j % 2] + ev
                    o_accs[j % 2] = o_accs[j % 2] + ov
                acc0 = e_accs[0] + e_accs[1]
                acc1 = o_accs[0] + o_accs[1]
                out_g[s, off * _RCHUNK + r, pl.ds(0, 16)] = acc0
                out_g[s, off * _RCHUNK + r, pl.ds(16, 16)] = acc1

            lax.fori_loop(0, _RCHUNK // 2, pair_body, 0)

        def process_group(g, s):
            base_r = wid * _RPW + g * _GRP_R

            @pl.when(g >= 2)
            def _():
                pltpu.make_async_copy(
                    out_g.at[s], out_hbm.at[pl.ds(0, _GRP_R)], osems[s]).wait()

            def step(cc, carry2):
                off_a = cc * 2
                gfire(off_a + 1, 1, s)
                gwait(0, s)
                compute(off_a, 0, s)

                @pl.when(cc < _GRP // 2 - 1)
                def _():
                    gfire(off_a + 2, 0, s)

                @pl.when((cc == _GRP // 2 - 1) & (g + 1 < _NGRP))
                def _():
                    iwait(1 - s)
                    gfire(0, 0, 1 - s)

                gwait(1, s)
                compute(off_a + 1, 1, s)
                return carry2

            lax.fori_loop(0, _GRP // 2, step, 0)
            pltpu.async_copy(out_g.at[s], out_hbm.at[pl.ds(base_r, _GRP_R)],
                             osems[s])

            @pl.when(g + 2 < _NGRP)
            def _():
                ifire(g + 2, s)

        ifire(0, 0)
        ifire(1, 1)
        iwait(0)
        gfire(0, 0, 0)

        def outer(gg, carry):
            process_group(gg * 2, 0)
            process_group(gg * 2 + 1, 1)
            return carry

        lax.fori_loop(0, _NGRP // 2, outer, 0)
        for s in range(2):
            pltpu.make_async_copy(
                out_g.at[s], out_hbm.at[pl.ds(0, _GRP_R)], osems[s]).wait()

    return k(table, idx, wgt)


def kernel(query, value, reference_points, spatial_shapes, level_start_index,
           W_value, b_value, W_offsets, b_offsets, W_attn, b_attn, W_out, b_out):
    qf = query.reshape(_BS * _NQ, _D)

    woxy_t = jnp.concatenate([W_offsets[0::2], W_offsets[1::2]], axis=0).T
    idx, wgt, table = _prep(
        query, value, W_value.T, b_value,
        woxy_t, W_attn.T,
        b_offsets[0::2], b_offsets[1::2], b_attn,
        reference_points.reshape(_BS, _NQ, 2 * _L),
    )
    idx = idx.reshape(4, _NSAMP // 128, 128)
    wgt = wgt.reshape(4, _NSAMP // 128, 128)
    table = table.reshape(_ROWS, _CH)

    msda = _sc_gather_accum(table, idx, wgt).reshape(_BS * _NQ, _D)

    p = np.arange(32)
    chan = np.where(p < 16, 2 * p, 2 * (p - 16) + 1)
    perm = (np.arange(_H)[:, None] * 32 + chan[None, :]).reshape(-1)
    out = _matmul_bias(msda, W_out.T[perm], b_out, residual=qf)
    return out.reshape(_BS, _NQ, _D)


# speedup vs baseline: 1.0129x; 1.0001x over previous
import functools
import math

import jax
import jax.numpy as jnp
from jax import lax
from jax.experimental import pallas as pl
from jax.experimental.pallas import tpu as pltpu
from jax.experimental.pallas import tpu_sc as plsc
import numpy as np

_SS = np.array([[64, 64], [32, 32], [16, 16], [8, 8]], dtype=np.int32)
_BS, _NQ, _D, _H, _L, _P = 2, 5440, 256, 8, 4, 4
_NV = int((_SS[:, 0] * _SS[:, 1]).sum())
_CH = _D // _H
_LSI = np.concatenate([[0], np.cumsum(_SS[:, 0] * _SS[:, 1])[:-1]]).astype(np.int32)

_ROWS = _BS * _NQ * _H
_NSAMP = _ROWS * _L * _P
_NWORK = 32
_RPW = _ROWS // _NWORK
_RCHUNK = 8
_NCHUNK = _RPW // _RCHUNK

_MM_BLK = 1088


def _mm_bias_kernel(x_ref, w_ref, b_ref, o_ref):
    o_ref[...] = (
        jnp.dot(x_ref[...], w_ref[...], preferred_element_type=jnp.float32)
        + b_ref[...]
    ).astype(o_ref.dtype)


def _mm_bias_res_kernel(x_ref, w_ref, b_ref, r_ref, o_ref):
    o_ref[...] = (
        jnp.dot(x_ref[...], w_ref[...], preferred_element_type=jnp.float32)
        + b_ref[...]
        + r_ref[...]
    )


def _matmul_bias(x, w_t, b, residual=None, out_dtype=jnp.float32):
    n, _ = x.shape
    m = w_t.shape[1]
    grid = (n // _MM_BLK,)
    in_specs = [
        pl.BlockSpec((_MM_BLK, x.shape[1]), lambda i: (i, 0)),
        pl.BlockSpec(w_t.shape, lambda i: (0, 0)),
        pl.BlockSpec((1, m), lambda i: (0, 0)),
    ]
    args = [x, w_t, b.reshape(1, m)]
    kern = _mm_bias_kernel
    if residual is not None:
        in_specs.append(pl.BlockSpec((_MM_BLK, m), lambda i: (i, 0)))
        args.append(residual)
        kern = _mm_bias_res_kernel
    return pl.pallas_call(
        kern,
        grid=grid,
        in_specs=in_specs,
        out_specs=pl.BlockSpec((_MM_BLK, m), lambda i: (i, 0)),
        out_shape=jax.ShapeDtypeStruct((n, m), out_dtype),
    )(*args)


_QB = 1088


def _prep_kernel(q_ref, v_ref, wv_ref, bv_ref, wox_ref, woy_ref, wat_ref,
                 box_ref, boy_ref, bat_ref, rp_ref, gones_ref,
                 idx_ref, wgt_ref, tbl_ref):
    b = pl.program_id(0)
    q = q_ref[0]

    tbl_ref[0] = (
        jnp.dot(v_ref[0], wv_ref[...], preferred_element_type=jnp.float32)
        + bv_ref[...]
    ).astype(jnp.bfloat16)

    so_x = jnp.dot(q, wox_ref[...], preferred_element_type=jnp.float32) + box_ref[...]
    so_y = jnp.dot(q, woy_ref[...], preferred_element_type=jnp.float32) + boy_ref[...]
    logits = jnp.dot(q, wat_ref[...], preferred_element_type=jnp.float32) + bat_ref[...]

    ex = jnp.exp(logits)
    gsum = jnp.dot(ex, gones_ref[...], preferred_element_type=jnp.float32)
    aw = ex / gsum

    lane = lax.broadcasted_iota(jnp.int32, (_QB, 128), 1)
    lsel = (lane // 4) % 4
    hsel = lane // 16

    def sel(vals, dtype):
        out = jnp.full(lane.shape, vals[3], dtype)
        for i in (2, 1, 0):
            out = jnp.where(lsel == i, dtype(vals[i]), out)
        return out

    wl_f = sel([float(_SS[i, 1]) for i in range(4)], jnp.float32)
    hl_f = sel([float(_SS[i, 0]) for i in range(4)], jnp.float32)
    wl_i = sel([int(_SS[i, 1]) for i in range(4)], jnp.int32)
    ls_i = sel([int(_LSI[i]) for i in range(4)], jnp.int32)

    rp = rp_ref[0]
    rx = jnp.zeros((_QB, 128), jnp.float32)
    ry = jnp.zeros((_QB, 128), jnp.float32)
    for l in range(_L):
        rx = jnp.where(lsel == l, rp[:, 2 * l:2 * l + 1], rx)
        ry = jnp.where(lsel == l, rp[:, 2 * l + 1:2 * l + 2], ry)

    gx = rx * wl_f - 0.5 + so_x
    gy = ry * hl_f - 0.5 + so_y
    x0 = jnp.floor(gx)
    y0 = jnp.floor(gy)
    wx1 = gx - x0
    wx0 = 1.0 - wx1
    wy1 = gy - y0
    wy0 = 1.0 - wy1

    base_b = b * _NV

    for c, (dx, dy, wx, wy) in enumerate(
        ((0, 0, wx0, wy0), (1, 0, wx1, wy0), (0, 1, wx0, wy1), (1, 1, wx1, wy1))
    ):
        xi = x0 + dx
        yi = y0 + dy
        valid = (
            (xi >= 0.0) & (xi <= wl_f - 1.0) & (yi >= 0.0) & (yi <= hl_f - 1.0)
        )
        xc = jnp.clip(xi, 0.0, wl_f - 1.0).astype(jnp.int32)
        yc = jnp.clip(yi, 0.0, hl_f - 1.0).astype(jnp.int32)
        row = (base_b + ls_i + yc * wl_i + xc) * _H + hsel
        w = aw * wx * wy * valid.astype(jnp.float32)
        idx_ref[c, 0] = row
        w16 = lax.bitcast_convert_type(w.astype(jnp.bfloat16), jnp.uint16)
        w32 = w16.astype(jnp.uint32)
        wgt_ref[c, 0] = lax.bitcast_convert_type(w32 | (w32 << 16), jnp.int32)


def _prep(query, value, wv_t, bv, wox_t, woy_t, wat_t, box, boy, bat, rp):
    grid = (_BS, _NQ // _QB)
    full = lambda shape: pl.BlockSpec(shape, lambda b, i: tuple(0 for _ in shape))
    in_specs = [
        pl.BlockSpec((1, _QB, _D), lambda b, i: (b, i, 0)),
        pl.BlockSpec((1, _QB, _D), lambda b, i: (b, i, 0)),
        full(wv_t.shape), full((1, _D)),
        full(wox_t.shape), full(woy_t.shape), full(wat_t.shape),
        full((1, 128)), full((1, 128)), full((1, 128)),
        pl.BlockSpec((1, _QB, 2 * _L), lambda b, i: (b, i, 0)),
        full((128, 128)),
    ]
    out_specs = [
        pl.BlockSpec((4, 1, _QB, 128), lambda b, i: (0, b, i, 0)),
        pl.BlockSpec((4, 1, _QB, 128), lambda b, i: (0, b, i, 0)),
        pl.BlockSpec((1, _QB, _D), lambda b, i: (b, i, 0)),
    ]
    pc = pl.pallas_call(
        _prep_kernel,
        grid=grid,
        in_specs=in_specs,
        out_specs=out_specs,
        out_shape=[
            jax.ShapeDtypeStruct((4, _BS, _NQ, 128), jnp.int32),
            jax.ShapeDtypeStruct((4, _BS, _NQ, 128), jnp.int32),
            jax.ShapeDtypeStruct((_BS, _NV, _D), jnp.bfloat16),
        ],
    )
    gones = jnp.asarray(np.kron(np.eye(_H, dtype=np.float32),
                                np.ones((16, 16), np.float32)))
    return pc(query, value, wv_t, bv.reshape(1, _D), wox_t, woy_t, wat_t,
              box.reshape(1, 128), boy.reshape(1, 128), bat.reshape(1, 128),
              rp, gones)


_GRP = 10
_NGRP = _NCHUNK // _GRP
_GRP_R = _GRP * _RCHUNK
_CPW = _NCHUNK


def _sc_gather_accum(table, idx, wgt):
    mesh = plsc.VectorSubcoreMesh(core_axis_name="c", subcore_axis_name="s")
    info = plsc.get_sparse_core_info()
    nc = info.num_cores

    @functools.partial(
        pl.kernel,
        mesh=mesh,
        out_type=jax.ShapeDtypeStruct((_ROWS, _CH), jnp.float32),
        compiler_params=pltpu.CompilerParams(
            use_tc_tiling_on_sc=False, needs_layout_passes=False),
        scratch_types=[
            pltpu.VMEM((2, 4, _GRP, 128), jnp.int32),
            pltpu.VMEM((2, 4, _GRP, 128), jnp.int32),
            pltpu.VMEM((2, 4, 128, _CH), jnp.bfloat16),
            pltpu.VMEM((2, _GRP_R, _CH), jnp.float32),
            pltpu.VMEM_SHARED((_ROWS, _CH), jnp.bfloat16),
            pltpu.SemaphoreType.DMA,
            pltpu.SemaphoreType.DMA,
            pltpu.SemaphoreType.DMA,
            pltpu.SemaphoreType.DMA,
            pltpu.SemaphoreType.DMA,
            pltpu.SemaphoreType.DMA,
        ],
    )
    def k(table_hbm, idx_hbm, wgt_hbm, out_hbm, idx_g, wgt_g, rows_v, out_g,
          table_sh, si0, si1, so0, so1, sb0, sb1):
        wid = lax.axis_index("s") * nc + lax.axis_index("c")
        sid = lax.axis_index("s")
        nst = _ROWS // 16
        pltpu.sync_copy(table_hbm.at[pl.ds(sid * nst, nst)],
                        table_sh.at[pl.ds(sid * nst, nst)])
        plsc.subcore_barrier()
        isems = (si0, si1)
        osems = (so0, so1)
        gsems = (sb0, sb1)

        def ifire(g, s):
            base_c = wid * _CPW + g * _GRP
            for c in range(4):
                pltpu.async_copy(
                    idx_hbm.at[c, pl.ds(base_c, _GRP)], idx_g.at[s, c], isems[s])
                pltpu.async_copy(
                    wgt_hbm.at[c, pl.ds(base_c, _GRP)], wgt_g.at[s, c], isems[s])

        def iwait(s):
            for c in range(4):
                pltpu.make_async_copy(
                    idx_hbm.at[c, pl.ds(0, _GRP)], idx_g.at[s, c], isems[s]).wait()
                pltpu.make_async_copy(
                    wgt_hbm.at[c, pl.ds(0, _GRP)], wgt_g.at[s, c], isems[s]).wait()

        def gfire(off, slot, s):
            for c in range(4):
                pltpu.async_copy(
                    table_sh.at[idx_g.at[s, c, off]],
                    rows_v.at[slot, c],
                    gsems[slot],
                )

        def gwait(slot, s):
            for c in range(4):
                pltpu.make_async_copy(
                    table_sh.at[idx_g.at[s, c, 0]],
                    rows_v.at[slot, c],
                    gsems[slot],
                ).wait()

        def compute(off, slot, s):
            def pair_body(t, carry2):
                for r in (t * 2, t * 2 + 1):
                    _one_row(off, slot, s, r)
                return carry2

            def _one_row(off, slot, s, r):
                j0 = r * (_L * _P)
                wvs = [wgt_g[s, c, off, pl.ds(j0, 16)] for c in range(4)]
                e_accs = [jnp.zeros((16,), jnp.float32) for _ in range(2)]
                o_accs = [jnp.zeros((16,), jnp.float32) for _ in range(2)]
                for j in range(_L * _P):
                    ps = []
                    for c in range(4):
                        w = wvs[c][j]
                        wspl = jnp.full((16,), w, jnp.int32)
                        wbf = plsc.bitcast(wspl, jnp.bfloat16)
                        ps.append(wbf * rows_v[slot, c, j0 + j])
                    p = (ps[0] + ps[1]) + (ps[2] + ps[3])
                    ev, ov = plsc.unpack(p, format=plsc.PackFormat.INTERLEAVED)
                    e_accs[j % 2] = e_accs[j % 2] + ev
                    o_accs[j % 2] = o_accs[j % 2] + ov
                acc0 = e_accs[0] + e_accs[1]
                acc1 = o_accs[0] + o_accs[1]
                out_g[s, off * _RCHUNK + r, pl.ds(0, 16)] = acc0
                out_g[s, off * _RCHUNK + r, pl.ds(16, 16)] = acc1

            lax.fori_loop(0, _RCHUNK // 2, pair_body, 0)

        def process_group(g, s):
            base_r = wid * _RPW + g * _GRP_R

            @pl.when(g >= 2)
            def _():
                pltpu.make_async_copy(
                    out_g.at[s], out_hbm.at[pl.ds(0, _GRP_R)], osems[s]).wait()

            def step(cc, carry2):
                off_a = cc * 2
                gfire(off_a + 1, 1, s)
                gwait(0, s)
                compute(off_a, 0, s)

                @pl.when(cc < _GRP // 2 - 1)
                def _():
                    gfire(off_a + 2, 0, s)

                @pl.when((cc == _GRP // 2 - 1) & (g + 1 < _NGRP))
                def _():
                    iwait(1 - s)
                    gfire(0, 0, 1 - s)

                gwait(1, s)
                compute(off_a + 1, 1, s)
                return carry2

            lax.fori_loop(0, _GRP // 2, step, 0)
            pltpu.async_copy(out_g.at[s], out_hbm.at[pl.ds(base_r, _GRP_R)],
                             osems[s])

            @pl.when(g + 2 < _NGRP)
            def _():
                ifire(g + 2, s)

        ifire(0, 0)
        ifire(1, 1)
        iwait(0)
        gfire(0, 0, 0)

        def outer(gg, carry):
            process_group(gg * 2, 0)
            process_group(gg * 2 + 1, 1)
            return carry

        lax.fori_loop(0, _NGRP // 2, outer, 0)
        for s in range(2):
            pltpu.make_async_copy(
                out_g.at[s], out_hbm.at[pl.ds(0, _GRP_R)], osems[s]).wait()

    return k(table, idx, wgt)


def kernel(query, value, reference_points, spatial_shapes, level_start_index,
           W_value, b_value, W_offsets, b_offsets, W_attn, b_attn, W_out, b_out):
    qf = query.reshape(_BS * _NQ, _D)

    idx, wgt, table = _prep(
        query, value, W_value.T, b_value,
        W_offsets[0::2].T, W_offsets[1::2].T, W_attn.T,
        b_offsets[0::2], b_offsets[1::2], b_attn,
        reference_points.reshape(_BS, _NQ, 2 * _L),
    )
    idx = idx.reshape(4, _NSAMP // 128, 128)
    wgt = wgt.reshape(4, _NSAMP // 128, 128)
    table = table.reshape(_ROWS, _CH)

    msda = _sc_gather_accum(table, idx, wgt).reshape(_BS * _NQ, _D)

    p = np.arange(32)
    chan = np.where(p < 16, 2 * p, 2 * (p - 16) + 1)
    perm = (np.arange(_H)[:, None] * 32 + chan[None, :]).reshape(-1)
    out = _matmul_bias(msda, W_out.T[perm], b_out, residual=qf)
    return out.reshape(_BS, _NQ, _D)
